# baseline (speedup 1.0000x reference)
import functools

import jax
import jax.numpy as jnp
from jax import lax
from jax.experimental import pallas as pl
from jax.experimental.pallas import tpu as pltpu
from jax.experimental.pallas import tpu_sc as plsc

B = 4096
L = 20
H = 64
DC = 8
DP = 8
BV = 96
GV = 104
BB = 2048
NB = B // BB

NC = 2
NS = 16
NW = NC * NS

CNT_N = 2 * B
_CNT_PW = CNT_N // NW


def _sc_gather_count(count_t, cnt_idx):
    mesh = plsc.VectorSubcoreMesh(core_axis_name="c", subcore_axis_name="s")

    @functools.partial(
        pl.kernel,
        mesh=mesh,
        compiler_params=pltpu.CompilerParams(use_tc_tiling_on_sc=False),
        out_type=jax.ShapeDtypeStruct((CNT_N, DC), jnp.float32),
        scratch_types=[
            pltpu.VMEM((_CNT_PW,), jnp.int32),
            pltpu.VMEM((_CNT_PW, DC), jnp.float32),
            pltpu.SemaphoreType.DMA,
        ],
    )
    def k(count_hbm, cnt_i_hbm, cnt_out, cnt_iv, cnt_rv, sem):
        wid = lax.axis_index("s") * NC + lax.axis_index("c")
        cb = wid * _CNT_PW
        pltpu.sync_copy(cnt_i_hbm.at[pl.ds(cb, _CNT_PW)], cnt_iv)
        pltpu.async_copy(count_hbm.at[cnt_iv], cnt_rv, sem).wait()
        pltpu.sync_copy(cnt_rv, cnt_out.at[pl.ds(cb, _CNT_PW)])

    return k(count_t, cnt_idx)


def _tc_body_a(fr_ref, to_ref, gi_ref, ii_ref, mr_ref, btT_ref,
               wihT_ref, baseT_ref, whh_ref, b2_ref,
               wgT_ref, genoT_ref, wbiT_ref, wbmT_ref,
               whfT_ref, whtT_ref, brc_ref, acc_ref, seqT_ref):
    f32 = jnp.float32
    bf16 = jnp.bfloat16
    seqT_ref[...] = jnp.concatenate(
        [jnp.transpose(fr_ref[...]), jnp.transpose(to_ref[...])], axis=1)
    wxT = jnp.dot(wihT_ref[...], baseT_ref[...],
                  preferred_element_type=f32).astype(bf16)
    whh = whh_ref[...].astype(bf16)
    b2 = b2_ref[...]
    iotaB = lax.broadcasted_iota(jnp.int32, (BV, 2 * BB), 0).astype(f32)

    h0 = jnp.zeros((H, 2 * BB), f32)
    c0 = jnp.zeros((H, 2 * BB), f32)

    def step(t, carry):
        h, c = carry
        oh = (iotaB == seqT_ref[t][None, :]).astype(bf16)
        g = (jnp.dot(wxT, oh, preferred_element_type=f32)
             + jnp.dot(whh, h.astype(bf16), preferred_element_type=f32) + b2)
        gi = jax.nn.sigmoid(g[0:H, :])
        gf = jax.nn.sigmoid(g[H:2 * H, :])
        gg = jnp.tanh(g[2 * H:3 * H, :])
        go = jax.nn.sigmoid(g[3 * H:4 * H, :])
        c = gf * c + gi * gg
        h = go * jnp.tanh(c)
        return h, c

    h, _ = lax.fori_loop(0, L, step, (h0, c0))

    acc = brc_ref[...] + jnp.dot(whfT_ref[...], h[:, 0:BB],
                                 preferred_element_type=f32)
    acc = acc + jnp.dot(whtT_ref[...], h[:, BB:2 * BB],
                        preferred_element_type=f32)
    wgeff = jnp.dot(wgT_ref[...], genoT_ref[...], preferred_element_type=f32)
    iotaG = lax.broadcasted_iota(jnp.int32, (GV, BB), 0).astype(f32)
    ohg = (iotaG == gi_ref[...]).astype(f32)
    acc = acc + jnp.dot(wgeff, ohg, preferred_element_type=f32)
    cIT = jnp.dot(wbiT_ref[...], btT_ref[...], preferred_element_type=f32)
    cMT = jnp.dot(wbmT_ref[...], btT_ref[...], preferred_element_type=f32)
    acc = acc + cIT[:, 0:1] + ii_ref[...] * (cIT[:, 1:2] - cIT[:, 0:1])
    acc = acc + cMT[:, 0:1] + mr_ref[...] * (cMT[:, 1:2] - cMT[:, 0:1])
    acc_ref[...] = acc


def _tc_body_b(acc_ref, cnt_ref, wf_ref, wv_ref, out_ref):
    f32 = jnp.float32
    o = (jnp.transpose(acc_ref[...])
         + jnp.dot(cnt_ref[0:B, :], wf_ref[...], preferred_element_type=f32)
         + jnp.dot(cnt_ref[B:2 * B, :], wv_ref[...],
                   preferred_element_type=f32))
    out_ref[...] = jnp.maximum(o, 0.0)


def kernel(gobyGenotypeIndex, isIndel, matchesReference, fromSequence, toSequence,
           genotypeCountForwardStrand, genotypeCountReverseStrand,
           geno_table, bool_table, base_table, count_table,
           W_ih, W_hh, b_ih, b_hh, W_red, b_red):
    i32 = jnp.int32
    f32 = jnp.float32

    cnt_idx = jnp.concatenate(
        [genotypeCountForwardStrand, genotypeCountReverseStrand],
        axis=0).astype(i32)
    count8 = jnp.pad(count_table.astype(f32), ((0, 0), (0, DC - 5)))
    emb_cnt = _sc_gather_count(count8, cnt_idx)

    frf = fromSequence.astype(f32)
    tof = toSequence.astype(f32)
    gidx = gobyGenotypeIndex.astype(f32).reshape(1, B)
    ii = isIndel.astype(f32).reshape(1, B)
    mr = matchesReference.astype(f32).reshape(1, B)

    wihT = jnp.pad(W_ih.astype(f32), ((0, 0), (0, DP - 6)))
    baseT = jnp.pad(base_table.astype(f32).T, ((0, DP - 6), (0, BV - 85)))
    whh = W_hh.astype(f32)
    b2 = (b_ih + b_hh).astype(f32).reshape(4 * H, 1)
    wr = W_red.astype(f32)
    wgT = jnp.pad(wr[0:4].T, ((0, 0), (0, DP - 4)))
    genoT = jnp.pad(geno_table.astype(f32).T,
                    ((0, DP - 4), (0, GV - 100)))
    wbiT = wr[4:6].T
    wbmT = wr[6:8].T
    whfT = wr[8:72].T
    whtT = wr[72:136].T
    wf = jnp.pad(wr[136:141], ((0, DC - 5), (0, 0)))
    wv = jnp.pad(wr[141:146], ((0, DC - 5), (0, 0)))
    brc = b_red.astype(f32).reshape(H, 1)
    btT = bool_table.astype(f32).T

    const = lambda shape: pl.BlockSpec(shape, lambda i: (0,) * len(shape))
    accT = pl.pallas_call(
        _tc_body_a,
        grid=(NB,),
        in_specs=[
            pl.BlockSpec((BB, L), lambda i: (i, 0)),
            pl.BlockSpec((BB, L), lambda i: (i, 0)),
            pl.BlockSpec((1, BB), lambda i: (0, i)),
            pl.BlockSpec((1, BB), lambda i: (0, i)),
            pl.BlockSpec((1, BB), lambda i: (0, i)),
            const((2, 2)),
            const((4 * H, DP)),
            const((DP, BV)),
            const((4 * H, H)),
            const((4 * H, 1)),
            const((H, DP)),
            const((DP, GV)),
            const((H, 2)),
            const((H, 2)),
            const((H, H)),
            const((H, H)),
            const((H, 1)),
        ],
        out_specs=pl.BlockSpec((H, BB), lambda i: (0, i)),
        out_shape=jax.ShapeDtypeStruct((H, B), f32),
        scratch_shapes=[pltpu.VMEM((L, 2 * BB), f32)],
    )(frf, tof, gidx, ii, mr, btT, wihT, baseT, whh, b2,
      wgT, genoT, wbiT, wbmT, whfT, whtT, brc)

    return pl.pallas_call(
        _tc_body_b,
        out_shape=jax.ShapeDtypeStruct((B, H), f32),
    )(accT, emb_cnt, wf, wv)


# speedup vs baseline: 1.0331x; 1.0331x over previous
import functools

import jax
import jax.numpy as jnp
from jax import lax
from jax.experimental import pallas as pl
from jax.experimental.pallas import tpu as pltpu
from jax.experimental.pallas import tpu_sc as plsc

B = 4096
L = 20
H = 64
DP = 8
BV = 96
GV = 104
BB = 2048
NB = B // BB

NC = 2
NS = 16
NW = NC * NS

CNT_N = 2 * B
_CNT_PW = CNT_N // NW


def _sc_gather_count(count_t, cnt_idx):
    mesh = plsc.VectorSubcoreMesh(core_axis_name="c", subcore_axis_name="s")

    @functools.partial(
        pl.kernel,
        mesh=mesh,
        compiler_params=pltpu.CompilerParams(use_tc_tiling_on_sc=False),
        out_type=jax.ShapeDtypeStruct((CNT_N, DP), jnp.float32),
        scratch_types=[
            pltpu.VMEM((_CNT_PW,), jnp.int32),
            pltpu.VMEM((_CNT_PW, DP), jnp.float32),
            pltpu.SemaphoreType.DMA,
        ],
    )
    def k(count_hbm, cnt_i_hbm, cnt_out, cnt_iv, cnt_rv, sem):
        wid = lax.axis_index("s") * NC + lax.axis_index("c")
        cb = wid * _CNT_PW
        pltpu.sync_copy(cnt_i_hbm.at[pl.ds(cb, _CNT_PW)], cnt_iv)
        pltpu.async_copy(count_hbm.at[cnt_iv], cnt_rv, sem).wait()
        pltpu.sync_copy(cnt_rv, cnt_out.at[pl.ds(cb, _CNT_PW)])

    return k(count_t, cnt_idx)


def _tc_body_a(seq_ref, gi_ref, ii_ref, mr_ref, btT_ref,
               wihT_ref, baseT_ref, whh_ref, b2_ref,
               wgT_ref, genoT_ref, wbiT_ref, wbmT_ref,
               whfT_ref, whtT_ref, brc_ref, acc_ref):
    f32 = jnp.float32
    bf16 = jnp.bfloat16
    wxT = jnp.dot(wihT_ref[...], baseT_ref[...],
                  preferred_element_type=f32).astype(bf16)
    whh = whh_ref[...].astype(bf16)
    b2 = b2_ref[...]
    iotaB = lax.broadcasted_iota(jnp.int32, (BV, 2 * BB), 0).astype(f32)

    h0 = jnp.zeros((H, 2 * BB), f32)
    c0 = jnp.zeros((H, 2 * BB), f32)

    def step(t, carry):
        h, c = carry
        oh = (iotaB == seq_ref[t][None, :]).astype(bf16)
        g = (jnp.dot(wxT, oh, preferred_element_type=f32)
             + jnp.dot(whh, h.astype(bf16), preferred_element_type=f32) + b2)
        gi = jax.nn.sigmoid(g[0:H, :])
        gf = jax.nn.sigmoid(g[H:2 * H, :])
        gg = jnp.tanh(g[2 * H:3 * H, :])
        go = jax.nn.sigmoid(g[3 * H:4 * H, :])
        c = gf * c + gi * gg
        h = go * jnp.tanh(c)
        return h, c

    h, _ = lax.fori_loop(0, L, step, (h0, c0))

    acc = brc_ref[...] + jnp.dot(whfT_ref[...], h[:, 0:BB],
                                 preferred_element_type=f32)
    acc = acc + jnp.dot(whtT_ref[...], h[:, BB:2 * BB],
                        preferred_element_type=f32)
    wgeff = jnp.dot(wgT_ref[...], genoT_ref[...], preferred_element_type=f32)
    iotaG = lax.broadcasted_iota(jnp.int32, (GV, BB), 0).astype(f32)
    ohg = (iotaG == gi_ref[...]).astype(f32)
    acc = acc + jnp.dot(wgeff, ohg, preferred_element_type=f32)
    cIT = jnp.dot(wbiT_ref[...], btT_ref[...], preferred_element_type=f32)
    cMT = jnp.dot(wbmT_ref[...], btT_ref[...], preferred_element_type=f32)
    acc = acc + cIT[:, 0:1] + ii_ref[...] * (cIT[:, 1:2] - cIT[:, 0:1])
    acc = acc + cMT[:, 0:1] + mr_ref[...] * (cMT[:, 1:2] - cMT[:, 0:1])
    acc_ref[...] = acc


def _tc_body_b(acc_ref, cntT_ref, wfT_ref, wvT_ref, out_ref):
    f32 = jnp.float32
    o = (acc_ref[...]
         + jnp.dot(wfT_ref[...], cntT_ref[0], preferred_element_type=f32)
         + jnp.dot(wvT_ref[...], cntT_ref[1], preferred_element_type=f32))
    out_ref[...] = jnp.maximum(o, 0.0)


def kernel(gobyGenotypeIndex, isIndel, matchesReference, fromSequence, toSequence,
           genotypeCountForwardStrand, genotypeCountReverseStrand,
           geno_table, bool_table, base_table, count_table,
           W_ih, W_hh, b_ih, b_hh, W_red, b_red):
    i32 = jnp.int32
    f32 = jnp.float32

    cnt_idx = jnp.concatenate(
        [genotypeCountForwardStrand, genotypeCountReverseStrand],
        axis=0).astype(i32)
    count8 = jnp.pad(count_table.astype(f32), ((0, 0), (0, DP - 5)))
    emb_cnt = _sc_gather_count(count8, cnt_idx)
    cntT = jnp.transpose(emb_cnt.reshape(2, B, DP), (0, 2, 1))

    fr = fromSequence.astype(f32).reshape(NB, BB, L)
    to = toSequence.astype(f32).reshape(NB, BB, L)
    seqf = jnp.stack([fr, to], axis=1).transpose(3, 0, 1, 2).reshape(L, 2 * B)

    gidx = gobyGenotypeIndex.astype(f32).reshape(1, B)
    ii = isIndel.astype(f32).reshape(1, B)
    mr = matchesReference.astype(f32).reshape(1, B)

    wihT = jnp.pad(W_ih.astype(f32), ((0, 0), (0, DP - 6)))
    baseT = jnp.pad(base_table.astype(f32).T, ((0, DP - 6), (0, BV - 85)))
    whh = W_hh.astype(f32)
    b2 = (b_ih + b_hh).astype(f32).reshape(4 * H, 1)
    wr = W_red.astype(f32)
    wgT = jnp.pad(wr[0:4].T, ((0, 0), (0, DP - 4)))
    genoT = jnp.pad(geno_table.astype(f32).T,
                    ((0, DP - 4), (0, GV - 100)))
    wbiT = wr[4:6].T
    wbmT = wr[6:8].T
    whfT = wr[8:72].T
    whtT = wr[72:136].T
    wfT = jnp.pad(wr[136:141].T, ((0, 0), (0, DP - 5)))
    wvT = jnp.pad(wr[141:146].T, ((0, 0), (0, DP - 5)))
    brc = b_red.astype(f32).reshape(H, 1)
    btT = bool_table.astype(f32).T

    const = lambda shape: pl.BlockSpec(shape, lambda i: (0,) * len(shape))
    accT = pl.pallas_call(
        _tc_body_a,
        grid=(NB,),
        in_specs=[
            pl.BlockSpec((L, 2 * BB), lambda i: (0, i)),
            pl.BlockSpec((1, BB), lambda i: (0, i)),
            pl.BlockSpec((1, BB), lambda i: (0, i)),
            pl.BlockSpec((1, BB), lambda i: (0, i)),
            const((2, 2)),
            const((4 * H, DP)),
            const((DP, BV)),
            const((4 * H, H)),
            const((4 * H, 1)),
            const((H, DP)),
            const((DP, GV)),
            const((H, 2)),
            const((H, 2)),
            const((H, H)),
            const((H, H)),
            const((H, 1)),
        ],
        out_specs=pl.BlockSpec((H, BB), lambda i: (0, i)),
        out_shape=jax.ShapeDtypeStruct((H, B), f32),
    )(seqf, gidx, ii, mr, btT, wihT, baseT, whh, b2,
      wgT, genoT, wbiT, wbmT, whfT, whtT, brc)

    outT = pl.pallas_call(
        _tc_body_b,
        out_shape=jax.ShapeDtypeStruct((H, B), f32),
    )(accT, cntT, wfT, wvT)
    return outT.T
